# Initial kernel scaffold; baseline (speedup 1.0000x reference)
import jax
import jax.numpy as jnp
from jax.experimental import pallas as pl


def kernel(cat, cont, emb_tables, cont_W):
    raise NotImplementedError("write your pallas kernel here")


# speedup vs baseline: 3.7425x; 3.7425x over previous
import jax
import jax.numpy as jnp
from jax import lax
from jax.experimental import pallas as pl
from jax.experimental.pallas import tpu as pltpu
from jax.experimental.pallas import tpu_sc as plsc

B, L, C, F, V, D = 4096, 50, 10, 4, 100000, 32
N = B * L
K = C + F
NC, NS = 2, 16
NW = NC * NS
TPW = N // NW
T = 128
NCHUNK = TPW // T
LN = 16
G = 128


def _body(cat_hbm, cont_hbm, tab_hbm, cw_hbm, out_hbm,
          catb, contb, cwb, idxb, dstb, rowb, crowb, sem):
    wid = lax.axis_index("s") * NC + lax.axis_index("c")
    iota = lax.iota(jnp.int32, LN)

    pltpu.sync_copy(cw_hbm, cwb)
    cwf = [[cwb[pl.ds(j * D + h * LN, LN)] for h in range(2)] for j in range(F)]

    def chunk_body(c, carry):
        gbase = wid * TPW + c * T
        rbase = gbase * C
        pltpu.sync_copy(cat_hbm.at[pl.ds(rbase, T * C)], catb)
        pltpu.sync_copy(cont_hbm.at[pl.ds(gbase * F, T * F)], contb)

        gk = gbase * K
        for q in range((T * C) // G):
            for i in range(G // LN):
                off = q * G + i * LN
                v = catb[pl.ds(off, LN)]
                r16 = off + iota
                fm = r16 % C
                td = lax.shift_right_logical(r16 * 6554, 16)
                idxb[pl.ds(i * LN, LN)] = v + fm * V
                dstb[pl.ds(i * LN, LN)] = gk + td * K + fm
            pltpu.async_copy(tab_hbm.at[idxb], rowb, sem).wait()
            pltpu.async_copy(rowb, out_hbm.at[dstb], sem).wait()

        def cont_body(ib, carry2):
            v16 = contb[pl.ds(ib * LN, LN)]
            for m in range(LN):
                r = ib * LN + m
                j = m % F
                s = v16[m]
                crowb[r, pl.ds(0, LN)] = cwf[j][0] * s
                crowb[r, pl.ds(LN, LN)] = cwf[j][1] * s
            return carry2

        lax.fori_loop(0, (T * F) // LN, cont_body, 0)
        for q in range((T * F) // G):
            for i in range(G // LN):
                off = q * G + i * LN
                r16 = off + iota
                dstb[pl.ds(i * LN, LN)] = (
                    gk + lax.shift_right_logical(r16, 2) * K + C + (r16 % F))
            pltpu.async_copy(crowb.at[pl.ds(q * G, G)], out_hbm.at[dstb],
                             sem).wait()
        return carry

    lax.fori_loop(0, NCHUNK, chunk_body, 0)


@jax.jit
def _run(cat2, contf, tabs, cont_W):
    mesh = plsc.VectorSubcoreMesh(core_axis_name="c", subcore_axis_name="s")
    return pl.kernel(
        _body,
        out_type=jax.ShapeDtypeStruct((N * K, D), jnp.float32),
        mesh=mesh,
        compiler_params=pltpu.CompilerParams(use_tc_tiling_on_sc=False),
        scratch_types=[
            pltpu.VMEM((T * C,), jnp.int32),
            pltpu.VMEM((T * F,), jnp.float32),
            pltpu.VMEM((F * D,), jnp.float32),
            pltpu.VMEM((G,), jnp.int32),
            pltpu.VMEM((G,), jnp.int32),
            pltpu.VMEM((G, D), jnp.float32),
            pltpu.VMEM((T * F, D), jnp.float32),
            pltpu.SemaphoreType.DMA,
        ],
    )(cat2, contf, tabs, cont_W)


def kernel(cat, cont, emb_tables, cont_W):
    cat2 = cat.reshape(N * C).astype(jnp.int32)
    contf = cont.reshape(N * F)
    tabs = emb_tables.reshape(C * V, D)
    cwf = cont_W.reshape(F * D)
    out = _run(cat2, contf, tabs, cwf)
    return out.reshape(N, K, D)


# speedup vs baseline: 4.3595x; 1.1649x over previous
import jax
import jax.numpy as jnp
from jax import lax
from jax.experimental import pallas as pl
from jax.experimental.pallas import tpu as pltpu
from jax.experimental.pallas import tpu_sc as plsc

B, L, C, F, V, D = 4096, 50, 10, 4, 100000, 32
N = B * L
K = C + F
NC, NS = 2, 16
NW = NC * NS
TPW = N // NW
T = 128
NCHUNK = TPW // T
LN = 16
G = 128
NGC = (T * C) // G
NGF = (T * F) // G


def _body(cat_hbm, cont_hbm, tab_hbm, cw_hbm, out_hbm,
          catb, contb, cwb, idxb, dstc, dstf, rowb, crowb, semG, semS):
    wid = lax.axis_index("s") * NC + lax.axis_index("c")
    iota = lax.iota(jnp.int32, LN)

    pltpu.sync_copy(cw_hbm, cwb)
    cwf = [[cwb[pl.ds(j * D + h * LN, LN)] for h in range(2)] for j in range(F)]

    def scatter_args():
        args = []
        for q in range(NGC):
            args.append((rowb.at[pl.ds(q * G, G)], out_hbm.at[dstc.at[q]]))
        for q in range(NGF):
            args.append((crowb.at[pl.ds(q * G, G)], out_hbm.at[dstf.at[q]]))
        return args

    def chunk_work(gbase):
        pltpu.sync_copy(cat_hbm.at[pl.ds(gbase * C, T * C)], catb)
        pltpu.sync_copy(cont_hbm.at[pl.ds(gbase * F, T * F)], contb)
        gk = gbase * K
        for i in range((T * C) // LN):
            off = i * LN
            v = catb[pl.ds(off, LN)]
            r16 = off + iota
            fm = r16 % C
            td = lax.shift_right_logical(r16 * 6554, 16)
            idxb[pl.ds(off, LN)] = v + fm * V
            dstc[i // (G // LN), pl.ds(off % G, LN)] = gk + td * K + fm
        handles = []
        for q in range(NGC):
            handles.append(pltpu.async_copy(
                tab_hbm.at[idxb.at[pl.ds(q * G, G)]],
                rowb.at[pl.ds(q * G, G)], semG))

        def cont_body(ib, carry):
            v16 = contb[pl.ds(ib * LN, LN)]
            for m in range(LN):
                r = ib * LN + m
                j = m % F
                sc = v16[m]
                crowb[r, pl.ds(0, LN)] = cwf[j][0] * sc
                crowb[r, pl.ds(LN, LN)] = cwf[j][1] * sc
            return carry

        lax.fori_loop(0, (T * F) // LN, cont_body, 0)
        for q in range(NGF):
            for i in range(G // LN):
                off = q * G + i * LN
                r16 = off + iota
                dstf[q, pl.ds(i * LN, LN)] = (
                    gk + lax.shift_right_logical(r16, 2) * K + C + (r16 % F))

        for h in handles:
            h.wait()
        for src, dst in scatter_args():
            pltpu.async_copy(src, dst, semS)

    def drain_scatters():
        for src, dst in scatter_args():
            pltpu.make_async_copy(src, dst, semS).wait()

    base0 = wid * TPW
    chunk_work(base0)

    def chunk_body(c, carry):
        drain_scatters()
        chunk_work(base0 + c * T)
        return carry

    lax.fori_loop(1, NCHUNK, chunk_body, 0)
    drain_scatters()


@jax.jit
def _run(cat2, contf, tabs, cont_W):
    mesh = plsc.VectorSubcoreMesh(core_axis_name="c", subcore_axis_name="s")
    return pl.kernel(
        _body,
        out_type=jax.ShapeDtypeStruct((N * K, D), jnp.float32),
        mesh=mesh,
        compiler_params=pltpu.CompilerParams(use_tc_tiling_on_sc=False),
        scratch_types=[
            pltpu.VMEM((T * C,), jnp.int32),
            pltpu.VMEM((T * F,), jnp.float32),
            pltpu.VMEM((F * D,), jnp.float32),
            pltpu.VMEM((T * C,), jnp.int32),
            pltpu.VMEM((NGC, G), jnp.int32),
            pltpu.VMEM((NGF, G), jnp.int32),
            pltpu.VMEM((T * C, D), jnp.float32),
            pltpu.VMEM((T * F, D), jnp.float32),
            pltpu.SemaphoreType.DMA,
            pltpu.SemaphoreType.DMA,
        ],
    )(cat2, contf, tabs, cont_W)


def kernel(cat, cont, emb_tables, cont_W):
    cat2 = cat.reshape(N * C).astype(jnp.int32)
    contf = cont.reshape(N * F)
    tabs = emb_tables.reshape(C * V, D)
    cwf = cont_W.reshape(F * D)
    out = _run(cat2, contf, tabs, cwf)
    return out.reshape(N, K, D)


# speedup vs baseline: 4.7154x; 1.0816x over previous
import jax
import jax.numpy as jnp
from jax import lax
from jax.experimental import pallas as pl
from jax.experimental.pallas import tpu as pltpu
from jax.experimental.pallas import tpu_sc as plsc

B, L, C, F, V, D = 4096, 50, 10, 4, 100000, 32
N = B * L
K = C + F
NC, NS = 2, 16
NW = NC * NS
LN = 16
G = 128
CH = 1280
NG = CH // G
EPW_CAT = N // NW
NCH_CAT = EPW_CAT // CH
EPW_CONT = (N * F) // NW
NCH_CONT = EPW_CONT // CH


def _body(cat_hbm, cont_hbm, tab_hbm, cw_hbm, out_hbm,
          catb, contb, cwb, idxb, dstb, rowb, semL, semG, semS):
    wid = lax.axis_index("s") * NC + lax.axis_index("c")
    iota = lax.iota(jnp.int32, LN)

    pltpu.sync_copy(cw_hbm, cwb)

    def drain_scatters():
        for q in range(NG):
            pltpu.make_async_copy(rowb.at[pl.ds(q * G, G)],
                                  out_hbm.at[dstb.at[q]], semS).wait()

    def fire_scatters():
        for q in range(NG):
            pltpu.async_copy(rowb.at[pl.ds(q * G, G)],
                             out_hbm.at[dstb.at[q]], semS)

    def cat_chunk(c, ch, guard):
        src_off = c * N + wid * EPW_CAT + ch * CH
        h = pltpu.async_copy(cat_hbm.at[pl.ds(src_off, CH)], catb, semL)
        if guard is None:
            drain_scatters()
        else:
            @pl.when(guard)
            def _():
                drain_scatters()
        h.wait()
        rr0 = wid * EPW_CAT + ch * CH
        for g in range(NG):
            e = rr0 + g * G
            l = lax.shift_right_logical(e, 12)
            bb = lax.bitwise_and(e, B - 1)
            dbase = l * K + c
            for i in range(G // LN):
                off = g * G + i * LN
                v = catb[pl.ds(off, LN)]
                idxb[pl.ds(off, LN)] = v + c * V
                b16 = bb + (i * LN + iota)
                dstb[g, pl.ds(i * LN, LN)] = b16 * (L * K) + dbase
        handles = []
        for q in range(NG):
            handles.append(pltpu.async_copy(
                tab_hbm.at[idxb.at[pl.ds(q * G, G)]],
                rowb.at[pl.ds(q * G, G)], semG))
        for h in handles:
            h.wait()
        fire_scatters()

    def cat_loop(ch, carry):
        cat_chunk(0, ch, ch > 0)
        for c in range(1, C):
            cat_chunk(c, ch, None)
        return carry

    lax.fori_loop(0, NCH_CAT, cat_loop, 0)

    def cont_chunk(ch, carry):
        src_off = wid * EPW_CONT + ch * CH
        h = pltpu.async_copy(cont_hbm.at[pl.ds(src_off, CH)], contb, semL)
        drain_scatters()
        h.wait()
        for g in range(NG):
            e = src_off + g * G
            l = lax.shift_right_logical(e, 14)
            f = lax.bitwise_and(lax.shift_right_logical(e, 12), F - 1)
            bb = lax.bitwise_and(e, B - 1)
            dbase = l * K + C + f
            cw_lo = cwb[pl.ds(f * D, LN)]
            cw_hi = cwb[pl.ds(f * D + LN, LN)]

            def blk(ib, carry2, g=g, cw_lo=cw_lo, cw_hi=cw_hi):
                v16 = contb[pl.ds(g * G + ib * LN, LN)]
                for m in range(LN):
                    r = g * G + ib * LN + m
                    sc = v16[m]
                    rowb[r, pl.ds(0, LN)] = cw_lo * sc
                    rowb[r, pl.ds(LN, LN)] = cw_hi * sc
                return carry2

            lax.fori_loop(0, G // LN, blk, 0)
            for i in range(G // LN):
                b16 = bb + (i * LN + iota)
                dstb[g, pl.ds(i * LN, LN)] = b16 * (L * K) + dbase
        fire_scatters()
        return carry

    lax.fori_loop(0, NCH_CONT, cont_chunk, 0)
    drain_scatters()


@jax.jit
def _run(catf, contf, tabs, cont_W):
    mesh = plsc.VectorSubcoreMesh(core_axis_name="c", subcore_axis_name="s")
    return pl.kernel(
        _body,
        out_type=jax.ShapeDtypeStruct((N * K, D), jnp.float32),
        mesh=mesh,
        compiler_params=pltpu.CompilerParams(use_tc_tiling_on_sc=False),
        scratch_types=[
            pltpu.VMEM((CH,), jnp.int32),
            pltpu.VMEM((CH,), jnp.float32),
            pltpu.VMEM((F * D,), jnp.float32),
            pltpu.VMEM((CH,), jnp.int32),
            pltpu.VMEM((NG, G), jnp.int32),
            pltpu.VMEM((CH, D), jnp.float32),
            pltpu.SemaphoreType.DMA,
            pltpu.SemaphoreType.DMA,
            pltpu.SemaphoreType.DMA,
        ],
    )(catf, contf, tabs, cont_W)


def kernel(cat, cont, emb_tables, cont_W):
    catf = jnp.transpose(cat, (2, 1, 0)).reshape(C * L * B).astype(jnp.int32)
    contf = jnp.transpose(cont, (1, 2, 0)).reshape(L * F * B)
    tabs = emb_tables.reshape(C * V, D)
    cwf = cont_W.reshape(F * D)
    out = _run(catf, contf, tabs, cwf)
    return out.reshape(N, K, D)


# speedup vs baseline: 5.2497x; 1.1133x over previous
import jax
import jax.numpy as jnp
from jax import lax
from jax.experimental import pallas as pl
from jax.experimental.pallas import tpu as pltpu
from jax.experimental.pallas import tpu_sc as plsc

B, L, C, F, V, D = 4096, 50, 10, 4, 100000, 32
N = B * L
K = C + F
NC, NS = 2, 16
NW = NC * NS
LN = 16
G = 128
CH = 1280
NG = CH // G
EPW = N // NW
NCH = EPW // CH


def _body(cat_hbm, cont_hbm, tab_hbm, cw_hbm, *outs_scr):
    outs = outs_scr[:K]
    catb, contb, cwb, idxb, dstb, rowb, semL, semG, semS = outs_scr[K:]
    wid = lax.axis_index("s") * NC + lax.axis_index("c")
    iota = lax.iota(jnp.int32, LN)

    pltpu.sync_copy(cw_hbm, cwb)
    cwf = [[cwb[pl.ds(j * D + h * LN, LN)] for h in range(2)] for j in range(F)]

    def drain_scatters(prev_out):
        for q in range(NG):
            pltpu.make_async_copy(rowb.at[pl.ds(q * G, G)],
                                  prev_out.at[dstb.at[q]], semS).wait()

    def fire_scatters(out):
        for q in range(NG):
            pltpu.async_copy(rowb.at[pl.ds(q * G, G)],
                             out.at[dstb.at[q]], semS)

    def dst_vectors(x0):
        for g in range(NG):
            e = x0 + g * G
            l = lax.shift_right_logical(e, 12)
            bb = lax.bitwise_and(e, B - 1)
            for i in range(G // LN):
                b16 = bb + (i * LN + iota)
                dstb[g, pl.ds(i * LN, LN)] = b16 * L + l

    def cat_chunk(c, ch, guard):
        x0 = wid * EPW + ch * CH
        h = pltpu.async_copy(cat_hbm.at[pl.ds(c * N + x0, CH)], catb, semL)
        prev = outs[c - 1] if c > 0 else outs[C - 1]
        if guard is None:
            drain_scatters(prev)
        else:
            @pl.when(guard)
            def _():
                drain_scatters(prev)
        h.wait()
        for i in range(CH // LN):
            off = i * LN
            idxb[pl.ds(off, LN)] = catb[pl.ds(off, LN)] + c * V
        dst_vectors(x0)
        handles = []
        for q in range(NG):
            handles.append(pltpu.async_copy(
                tab_hbm.at[idxb.at[pl.ds(q * G, G)]],
                rowb.at[pl.ds(q * G, G)], semG))
        for h2 in handles:
            h2.wait()
        fire_scatters(outs[c])

    def cat_loop(ch, carry):
        cat_chunk(0, ch, ch > 0)
        for c in range(1, C):
            cat_chunk(c, ch, None)
        return carry

    lax.fori_loop(0, NCH, cat_loop, 0)

    def cont_chunk(f, ch):
        x0 = wid * EPW + ch * CH
        lhandles = []
        for g in range(NG):
            e = x0 + g * G
            l = lax.shift_right_logical(e, 12)
            bb = lax.bitwise_and(e, B - 1)
            src = pl.multiple_of(l * (F * B) + f * B + bb, G)
            lhandles.append(pltpu.async_copy(
                cont_hbm.at[pl.ds(src, G)], contb.at[pl.ds(g * G, G)], semL))
        prev = outs[C + f - 1] if f > 0 else outs[C - 1]
        drain_scatters(prev)
        for h in lhandles:
            h.wait()
        cw_lo, cw_hi = cwf[f]

        def blk(ib, carry2):
            v16 = contb[pl.ds(ib * LN, LN)]
            for m in range(LN):
                r = ib * LN + m
                sc = v16[m]
                rowb[r, pl.ds(0, LN)] = cw_lo * sc
                rowb[r, pl.ds(LN, LN)] = cw_hi * sc
            return carry2

        lax.fori_loop(0, CH // LN, blk, 0)
        dst_vectors(x0)
        fire_scatters(outs[C + f])

    def cont_loop(ch, carry):
        for f in range(F):
            cont_chunk(f, ch)
        return carry

    lax.fori_loop(0, NCH, cont_loop, 0)
    drain_scatters(outs[K - 1])


@jax.jit
def _run(catf, contf, tabs, cont_W):
    mesh = plsc.VectorSubcoreMesh(core_axis_name="c", subcore_axis_name="s")
    return pl.kernel(
        _body,
        out_type=tuple(jax.ShapeDtypeStruct((N, D), jnp.float32)
                       for _ in range(K)),
        mesh=mesh,
        compiler_params=pltpu.CompilerParams(use_tc_tiling_on_sc=False),
        scratch_types=[
            pltpu.VMEM((CH,), jnp.int32),
            pltpu.VMEM((CH,), jnp.float32),
            pltpu.VMEM((F * D,), jnp.float32),
            pltpu.VMEM((CH,), jnp.int32),
            pltpu.VMEM((NG, G), jnp.int32),
            pltpu.VMEM((CH, D), jnp.float32),
            pltpu.SemaphoreType.DMA,
            pltpu.SemaphoreType.DMA,
            pltpu.SemaphoreType.DMA,
        ],
    )(catf, contf, tabs, cont_W)


def kernel(cat, cont, emb_tables, cont_W):
    catf = jnp.transpose(cat, (2, 1, 0)).reshape(C * L * B).astype(jnp.int32)
    contf = jnp.transpose(cont, (1, 2, 0)).reshape(L * F * B)
    tabs = emb_tables.reshape(C * V, D)
    cwf = cont_W.reshape(F * D)
    outs = _run(catf, contf, tabs, cwf)
    return jnp.stack(outs, axis=1)


# speedup vs baseline: 5.3764x; 1.0241x over previous
import jax
import jax.numpy as jnp
from jax import lax
from jax.experimental import pallas as pl
from jax.experimental.pallas import tpu as pltpu
from jax.experimental.pallas import tpu_sc as plsc

B, L, C, F, V, D = 4096, 50, 10, 4, 100000, 32
N = B * L
K = C + F
NC, NS = 2, 16
NW = NC * NS
LN = 16
G = 128
CH = 1280
NG = CH // G
EPW = N // NW
NCH = EPW // CH


def _dst_vectors(dstb, x0, iota):
    for g in range(NG):
        e = x0 + g * G
        l = lax.shift_right_logical(e, 12)
        bb = lax.bitwise_and(e, B - 1)
        for i in range(G // LN):
            b16 = bb + (i * LN + iota)
            dstb[g, pl.ds(i * LN, LN)] = b16 * L + l


def _cat_body(cat_hbm, tab_hbm, *outs_scr):
    outs = outs_scr[:C]
    catb, idxb, dstb, rowb, semL, semG, semS = outs_scr[C:]
    wid = lax.axis_index("s") * NC + lax.axis_index("c")
    iota = lax.iota(jnp.int32, LN)

    def drain_scatters(prev_out):
        for q in range(NG):
            pltpu.make_async_copy(rowb.at[pl.ds(q * G, G)],
                                  prev_out.at[dstb.at[q]], semS).wait()

    def cat_chunk(c, ch, guard):
        x0 = wid * EPW + ch * CH
        h = pltpu.async_copy(cat_hbm.at[pl.ds(c * N + x0, CH)], catb, semL)
        prev = outs[c - 1] if c > 0 else outs[C - 1]
        if guard is None:
            drain_scatters(prev)
        else:
            @pl.when(guard)
            def _():
                drain_scatters(prev)
        h.wait()
        for i in range(CH // LN):
            off = i * LN
            idxb[pl.ds(off, LN)] = catb[pl.ds(off, LN)] + c * V
        _dst_vectors(dstb, x0, iota)
        handles = []
        for q in range(NG):
            handles.append(pltpu.async_copy(
                tab_hbm.at[idxb.at[pl.ds(q * G, G)]],
                rowb.at[pl.ds(q * G, G)], semG))
        for h2 in handles:
            h2.wait()
        for q in range(NG):
            pltpu.async_copy(rowb.at[pl.ds(q * G, G)],
                             outs[c].at[dstb.at[q]], semS)

    def cat_loop(ch, carry):
        cat_chunk(0, ch, ch > 0)
        for c in range(1, C):
            cat_chunk(c, ch, None)
        return carry

    lax.fori_loop(0, NCH, cat_loop, 0)
    drain_scatters(outs[C - 1])


def _cont_body(cont_hbm, cw_hbm, *outs_scr):
    outs = outs_scr[:F]
    contb, cwb, dstb, rowb, semL, semS = outs_scr[F:]
    wid = lax.axis_index("s") * NC + lax.axis_index("c")
    iota = lax.iota(jnp.int32, LN)

    pltpu.sync_copy(cw_hbm, cwb)
    cwf = [[cwb[pl.ds(j * D + h * LN, LN)] for h in range(2)] for j in range(F)]

    def drain_scatters(prev_out):
        for q in range(NG):
            pltpu.make_async_copy(rowb.at[pl.ds(q * G, G)],
                                  prev_out.at[dstb.at[q]], semS).wait()

    def cont_chunk(f, ch, guard):
        x0 = wid * EPW + ch * CH
        lhandles = []
        for g in range(NG):
            e = x0 + g * G
            l = lax.shift_right_logical(e, 12)
            bb = lax.bitwise_and(e, B - 1)
            src = pl.multiple_of(l * (F * B) + f * B + bb, G)
            lhandles.append(pltpu.async_copy(
                cont_hbm.at[pl.ds(src, G)], contb.at[pl.ds(g * G, G)], semL))
        prev = outs[f - 1] if f > 0 else outs[F - 1]
        if guard is None:
            drain_scatters(prev)
        else:
            @pl.when(guard)
            def _():
                drain_scatters(prev)
        for h in lhandles:
            h.wait()
        cw_lo, cw_hi = cwf[f]

        def blk(ib, carry2):
            v16 = contb[pl.ds(ib * LN, LN)]
            for m in range(LN):
                r = ib * LN + m
                sc = v16[m]
                rowb[r, pl.ds(0, LN)] = cw_lo * sc
                rowb[r, pl.ds(LN, LN)] = cw_hi * sc
            return carry2

        lax.fori_loop(0, CH // LN, blk, 0)
        _dst_vectors(dstb, x0, iota)
        for q in range(NG):
            pltpu.async_copy(rowb.at[pl.ds(q * G, G)],
                             outs[f].at[dstb.at[q]], semS)

    def cont_loop(ch, carry):
        cont_chunk(0, ch, ch > 0)
        for f in range(1, F):
            cont_chunk(f, ch, None)
        return carry

    lax.fori_loop(0, NCH, cont_loop, 0)
    drain_scatters(outs[F - 1])


@jax.jit
def _run(catf, contf, tabs, cont_W):
    mesh = plsc.VectorSubcoreMesh(core_axis_name="c", subcore_axis_name="s")
    cont_outs = pl.kernel(
        _cont_body,
        out_type=tuple(jax.ShapeDtypeStruct((N, D), jnp.float32)
                       for _ in range(F)),
        mesh=mesh,
        compiler_params=pltpu.CompilerParams(use_tc_tiling_on_sc=False),
        scratch_types=[
            pltpu.VMEM((CH,), jnp.float32),
            pltpu.VMEM((F * D,), jnp.float32),
            pltpu.VMEM((NG, G), jnp.int32),
            pltpu.VMEM((CH, D), jnp.float32),
            pltpu.SemaphoreType.DMA,
            pltpu.SemaphoreType.DMA,
        ],
    )(contf, cont_W)
    cat_outs = pl.kernel(
        _cat_body,
        out_type=tuple(jax.ShapeDtypeStruct((N, D), jnp.float32)
                       for _ in range(C)),
        mesh=mesh,
        compiler_params=pltpu.CompilerParams(use_tc_tiling_on_sc=False),
        scratch_types=[
            pltpu.VMEM((CH,), jnp.int32),
            pltpu.VMEM((CH,), jnp.int32),
            pltpu.VMEM((NG, G), jnp.int32),
            pltpu.VMEM((CH, D), jnp.float32),
            pltpu.SemaphoreType.DMA,
            pltpu.SemaphoreType.DMA,
            pltpu.SemaphoreType.DMA,
        ],
    )(catf, tabs)
    return cat_outs + cont_outs


def kernel(cat, cont, emb_tables, cont_W):
    catf = jnp.transpose(cat, (2, 1, 0)).reshape(C * L * B).astype(jnp.int32)
    contf = jnp.transpose(cont, (1, 2, 0)).reshape(L * F * B)
    tabs = emb_tables.reshape(C * V, D)
    cwf = cont_W.reshape(F * D)
    outs = _run(catf, contf, tabs, cwf)
    return jnp.stack(outs, axis=1)


# speedup vs baseline: 5.5767x; 1.0372x over previous
import jax
import jax.numpy as jnp
from jax import lax
from jax.experimental import pallas as pl
from jax.experimental.pallas import tpu as pltpu
from jax.experimental.pallas import tpu_sc as plsc

B, L, C, F, V, D = 4096, 50, 10, 4, 100000, 32
N = B * L
K = C + F
NC, NS = 2, 16
NW = NC * NS
LN = 16
G = 128
CH = 1280
NG = CH // G
EPW = N // NW
NCH = EPW // CH


def _dst_vectors(dstb, x0, iota):
    for g in range(NG):
        e = x0 + g * G
        l = lax.shift_right_logical(e, 12)
        bb = lax.bitwise_and(e, B - 1)
        for i in range(G // LN):
            b16 = bb + (i * LN + iota)
            dstb[g, pl.ds(i * LN, LN)] = b16 * L + l


def _make_cat_body(fields):
    nf = len(fields)

    def _cat_body(cat_hbm, tab_hbm, *outs_scr):
        outs = outs_scr[:nf]
        catb, idxb, dstb, rowb, semL, semG, semS = outs_scr[nf:]
        wid = lax.axis_index("s") * NC + lax.axis_index("c")
        iota = lax.iota(jnp.int32, LN)

        def drain_scatters(prev_out):
            for q in range(NG):
                pltpu.make_async_copy(rowb.at[pl.ds(q * G, G)],
                                      prev_out.at[dstb.at[q]], semS).wait()

        def cat_chunk(ci, ch, guard):
            c = fields[ci]
            x0 = wid * EPW + ch * CH
            h = pltpu.async_copy(cat_hbm.at[pl.ds(c * N + x0, CH)], catb,
                                 semL)
            prev = outs[ci - 1] if ci > 0 else outs[nf - 1]
            if guard is None:
                drain_scatters(prev)
            else:
                @pl.when(guard)
                def _():
                    drain_scatters(prev)
            h.wait()
            for i in range(CH // LN):
                off = i * LN
                idxb[pl.ds(off, LN)] = catb[pl.ds(off, LN)] + c * V
            _dst_vectors(dstb, x0, iota)
            handles = []
            for q in range(NG):
                handles.append(pltpu.async_copy(
                    tab_hbm.at[idxb.at[pl.ds(q * G, G)]],
                    rowb.at[pl.ds(q * G, G)], semG))
            for h2 in handles:
                h2.wait()
            for q in range(NG):
                pltpu.async_copy(rowb.at[pl.ds(q * G, G)],
                                 outs[ci].at[dstb.at[q]], semS)

        def cat_loop(ch, carry):
            cat_chunk(0, ch, ch > 0)
            for ci in range(1, nf):
                cat_chunk(ci, ch, None)
            return carry

        lax.fori_loop(0, NCH, cat_loop, 0)
        drain_scatters(outs[nf - 1])

    return _cat_body


def _cont_body(cont_hbm, cw_hbm, *outs_scr):
    outs = outs_scr[:F]
    contb, cwb, dstb, rowb, semL, semS = outs_scr[F:]
    wid = lax.axis_index("s") * NC + lax.axis_index("c")
    iota = lax.iota(jnp.int32, LN)

    pltpu.sync_copy(cw_hbm, cwb)
    cwf = [[cwb[pl.ds(j * D + h * LN, LN)] for h in range(2)] for j in range(F)]

    def drain_scatters(prev_out):
        for q in range(NG):
            pltpu.make_async_copy(rowb.at[pl.ds(q * G, G)],
                                  prev_out.at[dstb.at[q]], semS).wait()

    def cont_chunk(f, ch, guard):
        x0 = wid * EPW + ch * CH
        lhandles = []
        for g in range(NG):
            e = x0 + g * G
            l = lax.shift_right_logical(e, 12)
            bb = lax.bitwise_and(e, B - 1)
            src = pl.multiple_of(l * (F * B) + f * B + bb, G)
            lhandles.append(pltpu.async_copy(
                cont_hbm.at[pl.ds(src, G)], contb.at[pl.ds(g * G, G)], semL))
        prev = outs[f - 1] if f > 0 else outs[F - 1]
        if guard is None:
            drain_scatters(prev)
        else:
            @pl.when(guard)
            def _():
                drain_scatters(prev)
        for h in lhandles:
            h.wait()
        cw_lo, cw_hi = cwf[f]

        def blk(ib, carry2):
            v16 = contb[pl.ds(ib * LN, LN)]
            for m in range(LN):
                r = ib * LN + m
                sc = v16[m]
                rowb[r, pl.ds(0, LN)] = cw_lo * sc
                rowb[r, pl.ds(LN, LN)] = cw_hi * sc
            return carry2

        lax.fori_loop(0, CH // LN, blk, 0)
        _dst_vectors(dstb, x0, iota)
        for q in range(NG):
            pltpu.async_copy(rowb.at[pl.ds(q * G, G)],
                             outs[f].at[dstb.at[q]], semS)

    def cont_loop(ch, carry):
        cont_chunk(0, ch, ch > 0)
        for f in range(1, F):
            cont_chunk(f, ch, None)
        return carry

    lax.fori_loop(0, NCH, cont_loop, 0)
    drain_scatters(outs[F - 1])


@jax.jit
def _run(catf, contf, tabs, cont_W):
    mesh = plsc.VectorSubcoreMesh(core_axis_name="c", subcore_axis_name="s")
    cont_outs = pl.kernel(
        _cont_body,
        out_type=tuple(jax.ShapeDtypeStruct((N, D), jnp.float32)
                       for _ in range(F)),
        mesh=mesh,
        compiler_params=pltpu.CompilerParams(use_tc_tiling_on_sc=False),
        scratch_types=[
            pltpu.VMEM((CH,), jnp.float32),
            pltpu.VMEM((F * D,), jnp.float32),
            pltpu.VMEM((NG, G), jnp.int32),
            pltpu.VMEM((CH, D), jnp.float32),
            pltpu.SemaphoreType.DMA,
            pltpu.SemaphoreType.DMA,
        ],
    )(contf, cont_W)
    cat_outs = []
    for fields in ((0, 1), (2, 3, 4, 5), (6, 7, 8, 9)):
        cat_outs.extend(pl.kernel(
            _make_cat_body(fields),
            out_type=tuple(jax.ShapeDtypeStruct((N, D), jnp.float32)
                           for _ in fields),
            mesh=mesh,
            compiler_params=pltpu.CompilerParams(use_tc_tiling_on_sc=False),
            scratch_types=[
                pltpu.VMEM((CH,), jnp.int32),
                pltpu.VMEM((CH,), jnp.int32),
                pltpu.VMEM((NG, G), jnp.int32),
                pltpu.VMEM((CH, D), jnp.float32),
                pltpu.SemaphoreType.DMA,
                pltpu.SemaphoreType.DMA,
                pltpu.SemaphoreType.DMA,
            ],
        )(catf, tabs))
    return tuple(cat_outs) + cont_outs


def kernel(cat, cont, emb_tables, cont_W):
    catf = jnp.transpose(cat, (2, 1, 0)).reshape(C * L * B).astype(jnp.int32)
    contf = jnp.transpose(cont, (1, 2, 0)).reshape(L * F * B)
    tabs = emb_tables.reshape(C * V, D)
    cwf = cont_W.reshape(F * D)
    outs = _run(catf, contf, tabs, cwf)
    return jnp.stack(outs, axis=1)


# speedup vs baseline: 5.5878x; 1.0020x over previous
import jax
import jax.numpy as jnp
from jax import lax
from jax.experimental import pallas as pl
from jax.experimental.pallas import tpu as pltpu
from jax.experimental.pallas import tpu_sc as plsc

B, L, C, F, V, D = 4096, 50, 10, 4, 100000, 32
N = B * L
K = C + F
NC, NS = 2, 16
NW = NC * NS
LN = 16
G = 128
CH = 1280
NG = CH // G
EPW = N // NW
NCH = EPW // CH


def _dst_vectors(dstb, x0, iota):
    for g in range(NG):
        e = x0 + g * G
        l = lax.shift_right_logical(e, 12)
        bb = lax.bitwise_and(e, B - 1)
        for i in range(G // LN):
            b16 = bb + (i * LN + iota)
            dstb[g, pl.ds(i * LN, LN)] = b16 * L + l


def _make_cat_body(fields):
    nf = len(fields)

    def _cat_body(cat_hbm, tab_hbm, *outs_scr):
        outs = outs_scr[:nf]
        catb, idxb, dstb, rowb, semL, semG, semS = outs_scr[nf:]
        wid = lax.axis_index("s") * NC + lax.axis_index("c")
        iota = lax.iota(jnp.int32, LN)

        def drain_scatters(prev_out):
            for q in range(NG):
                pltpu.make_async_copy(rowb.at[pl.ds(q * G, G)],
                                      prev_out.at[dstb.at[q]], semS).wait()

        def cat_chunk(ci, ch, guard):
            c = fields[ci]
            x0 = wid * EPW + ch * CH
            h = pltpu.async_copy(cat_hbm.at[pl.ds(c * N + x0, CH)], catb,
                                 semL)
            prev = outs[ci - 1] if ci > 0 else outs[nf - 1]
            if guard is None:
                drain_scatters(prev)
            else:
                @pl.when(guard)
                def _():
                    drain_scatters(prev)
            h.wait()
            for i in range(CH // LN):
                off = i * LN
                idxb[pl.ds(off, LN)] = catb[pl.ds(off, LN)] + c * V
            _dst_vectors(dstb, x0, iota)
            handles = []
            for q in range(NG):
                handles.append(pltpu.async_copy(
                    tab_hbm.at[idxb.at[pl.ds(q * G, G)]],
                    rowb.at[pl.ds(q * G, G)], semG))
            for h2 in handles:
                h2.wait()
            for q in range(NG):
                pltpu.async_copy(rowb.at[pl.ds(q * G, G)],
                                 outs[ci].at[dstb.at[q]], semS)

        def cat_loop(ch, carry):
            cat_chunk(0, ch, ch > 0)
            for ci in range(1, nf):
                cat_chunk(ci, ch, None)
            return carry

        lax.fori_loop(0, NCH, cat_loop, 0)
        drain_scatters(outs[nf - 1])

    return _cat_body


def _cont_body(cont_hbm, cw_hbm, *outs_scr):
    outs = outs_scr[:F]
    contb, cwb, dstb, rowb, semL, semS = outs_scr[F:]
    wid = lax.axis_index("s") * NC + lax.axis_index("c")
    iota = lax.iota(jnp.int32, LN)

    pltpu.sync_copy(cw_hbm, cwb)
    cwf = [[cwb[pl.ds(j * D + h * LN, LN)] for h in range(2)] for j in range(F)]

    def drain_scatters(prev_out):
        for q in range(NG):
            pltpu.make_async_copy(rowb.at[pl.ds(q * G, G)],
                                  prev_out.at[dstb.at[q]], semS).wait()

    def cont_chunk(f, ch, guard):
        x0 = wid * EPW + ch * CH
        lhandles = []
        for g in range(NG):
            e = x0 + g * G
            l = lax.shift_right_logical(e, 12)
            bb = lax.bitwise_and(e, B - 1)
            src = pl.multiple_of(l * (F * B) + f * B + bb, G)
            lhandles.append(pltpu.async_copy(
                cont_hbm.at[pl.ds(src, G)], contb.at[pl.ds(g * G, G)], semL))
        prev = outs[f - 1] if f > 0 else outs[F - 1]
        if guard is None:
            drain_scatters(prev)
        else:
            @pl.when(guard)
            def _():
                drain_scatters(prev)
        for h in lhandles:
            h.wait()
        cw_lo, cw_hi = cwf[f]

        def blk(ib, carry2):
            v16 = contb[pl.ds(ib * LN, LN)]
            for m in range(LN):
                r = ib * LN + m
                sc = v16[m]
                rowb[r, pl.ds(0, LN)] = cw_lo * sc
                rowb[r, pl.ds(LN, LN)] = cw_hi * sc
            return carry2

        lax.fori_loop(0, CH // LN, blk, 0)
        _dst_vectors(dstb, x0, iota)
        for q in range(NG):
            pltpu.async_copy(rowb.at[pl.ds(q * G, G)],
                             outs[f].at[dstb.at[q]], semS)

    def cont_loop(ch, carry):
        cont_chunk(0, ch, ch > 0)
        for f in range(1, F):
            cont_chunk(f, ch, None)
        return carry

    lax.fori_loop(0, NCH, cont_loop, 0)
    drain_scatters(outs[F - 1])


@jax.jit
def _run(catf, contf, tabs, cont_W):
    mesh = plsc.VectorSubcoreMesh(core_axis_name="c", subcore_axis_name="s")
    cont_outs = pl.kernel(
        _cont_body,
        out_type=tuple(jax.ShapeDtypeStruct((N, D), jnp.float32)
                       for _ in range(F)),
        mesh=mesh,
        compiler_params=pltpu.CompilerParams(use_tc_tiling_on_sc=False),
        scratch_types=[
            pltpu.VMEM((CH,), jnp.float32),
            pltpu.VMEM((F * D,), jnp.float32),
            pltpu.VMEM((NG, G), jnp.int32),
            pltpu.VMEM((CH, D), jnp.float32),
            pltpu.SemaphoreType.DMA,
            pltpu.SemaphoreType.DMA,
        ],
    )(contf, cont_W)
    cat_outs = []
    for fields in ((0, 1), (2, 3, 4, 5), (6, 7, 8, 9)):
        cat_outs.extend(pl.kernel(
            _make_cat_body(fields),
            out_type=tuple(jax.ShapeDtypeStruct((N, D), jnp.float32)
                           for _ in fields),
            mesh=mesh,
            compiler_params=pltpu.CompilerParams(use_tc_tiling_on_sc=False),
            scratch_types=[
                pltpu.VMEM((CH,), jnp.int32),
                pltpu.VMEM((CH,), jnp.int32),
                pltpu.VMEM((NG, G), jnp.int32),
                pltpu.VMEM((CH, D), jnp.float32),
                pltpu.SemaphoreType.DMA,
                pltpu.SemaphoreType.DMA,
                pltpu.SemaphoreType.DMA,
            ],
        )(catf, tabs))
    return tuple(cat_outs) + cont_outs


def kernel(cat, cont, emb_tables, cont_W):
    catf = jnp.transpose(cat, (2, 1, 0)).reshape(C * L * B).astype(jnp.int32)
    contf = jnp.transpose(cont, (1, 2, 0)).reshape(L * F * B)
    tabs = emb_tables.reshape(C * V, D)
    cwf = cont_W.reshape(F * D)
    outs = _run(catf, contf, tabs, cwf)
    return jnp.concatenate(
        [jnp.stack(outs[:7], axis=1), jnp.stack(outs[7:], axis=1)], axis=1)
